# Initial kernel scaffold; baseline (speedup 1.0000x reference)
import jax
import jax.numpy as jnp
from jax.experimental import pallas as pl


def kernel(x, emb_0, emb_1, emb_2, emb_3, emb_4, emb_5, emb_6, emb_7, emb_8, emb_9, emb_10, emb_11, emb_12, emb_13, emb_14, emb_15, emb_16, emb_17, emb_18, emb_19, emb_20, emb_21, emb_22, emb_23, emb_24, emb_25):
    raise NotImplementedError("write your pallas kernel here")


# speedup vs baseline: 1.0241x; 1.0241x over previous
import jax
import jax.numpy as jnp
from jax import lax
from jax.experimental import pallas as pl
from jax.experimental.pallas import tpu as pltpu
from jax.experimental.pallas import tpu_sc as plsc

BATCH = 16384
INPUT_DIM = 39
N_DENSE = 13
N_CAT = 26
EMB_DIM = 16
OUT_DIM = N_DENSE + N_CAT * EMB_DIM
PAD = 3
PADDED = PAD + OUT_DIM

NUM_CORES = 2
NUM_SUBCORES = 16
LANES = 16
NW = NUM_CORES * NUM_SUBCORES
BPW = BATCH // NW


def _encoder_body(xT, *refs):
    tables = refs[:N_CAT]
    out = refs[N_CAT]
    idx2, dload, dbuf, gbuf, sem = refs[N_CAT + 1:]

    wid = lax.axis_index("s") * NUM_CORES + lax.axis_index("c")
    base = pl.multiple_of(wid * jnp.int32(BPW), BPW)

    pltpu.sync_copy(xT.at[pl.ds(N_DENSE, N_CAT), pl.ds(base, BPW)], idx2)
    for j in range(N_DENSE):
        pltpu.sync_copy(
            xT.at[jnp.int32(j), pl.ds(base, BPW)],
            dload.at[pl.ds(j * BPW, BPW)],
        )

    for i in range(N_CAT):
        pltpu.async_copy(tables[i].at[idx2.at[jnp.int32(i)]], gbuf, sem).wait()
        pltpu.sync_copy(
            gbuf, out.at[pl.ds(base, BPW), pl.ds(PAD + N_DENSE + i * EMB_DIM, EMB_DIM)]
        )

    lane = lax.iota(jnp.int32, LANES)
    stride = jnp.maximum(lane - PAD, 0) * jnp.int32(BPW)

    def grp(c, carry):
        r0 = c * jnp.int32(LANES)
        for off in range(LANES):
            r = r0 + jnp.int32(off)
            vals = plsc.load_gather(dload, [stride + r]).astype(jnp.float32)
            dbuf[r, :] = vals
        return carry

    lax.fori_loop(jnp.int32(0), jnp.int32(BPW // LANES), grp, jnp.int32(0))
    pltpu.sync_copy(dbuf, out.at[pl.ds(base, BPW), pl.ds(0, PAD + N_DENSE)])


@jax.jit
def _encode(xT, *tables):
    mesh = plsc.VectorSubcoreMesh(core_axis_name="c", subcore_axis_name="s")
    padded = pl.kernel(
        _encoder_body,
        mesh=mesh,
        out_type=jax.ShapeDtypeStruct((BATCH, PADDED), jnp.float32),
        scratch_types=[
            pltpu.VMEM((N_CAT, BPW), jnp.int32),
            pltpu.VMEM((N_DENSE * BPW,), jnp.int32),
            pltpu.VMEM((BPW, PAD + N_DENSE), jnp.float32),
            pltpu.VMEM((BPW, EMB_DIM), jnp.float32),
            pltpu.SemaphoreType.DMA,
        ],
        compiler_params=pltpu.CompilerParams(
            use_tc_tiling_on_sc=False, needs_layout_passes=False
        ),
    )(xT, *tables)
    return padded[:, PAD:]


def kernel(x, emb_0, emb_1, emb_2, emb_3, emb_4, emb_5, emb_6, emb_7, emb_8,
           emb_9, emb_10, emb_11, emb_12, emb_13, emb_14, emb_15, emb_16,
           emb_17, emb_18, emb_19, emb_20, emb_21, emb_22, emb_23, emb_24,
           emb_25):
    xT = jnp.asarray(x, jnp.int32).T
    return _encode(xT, emb_0, emb_1, emb_2, emb_3, emb_4, emb_5, emb_6, emb_7,
                   emb_8, emb_9, emb_10, emb_11, emb_12, emb_13, emb_14,
                   emb_15, emb_16, emb_17, emb_18, emb_19, emb_20, emb_21,
                   emb_22, emb_23, emb_24, emb_25)
